# baseline (speedup 1.0000x reference)
import functools

import jax
import jax.numpy as jnp
from jax import lax
from jax.experimental import pallas as pl
from jax.experimental.pallas import tpu as pltpu
from jax.experimental.pallas import tpu_sc as plsc

F = 26
V = 100000
D = 32
B = 16384
LOG2_B = 14

NC = 2
NS = 16
NW = NC * NS
TOTAL_ROWS = F * B
RPW = TOTAL_ROWS // NW
CHUNK = 416
NCHUNK = RPW // CHUNK


def _sc_gather_body(tables_hbm, idx_hbm, out_hbm,
                    idx_a, idx_b, wide_a, wide_b, sem_a, sem_b):
    wid = lax.axis_index("s") * NC + lax.axis_index("c")
    base = pl.multiple_of(wid * RPW, CHUNK)
    idx_bufs = (idx_a, idx_b)
    wide_bufs = (wide_a, wide_b)
    sems = (sem_a, sem_b)

    def stage_indices(c, k):
        pltpu.sync_copy(idx_hbm.at[pl.ds(base + c * CHUNK, CHUNK)], idx_bufs[k])

        def fix(i, _):
            off = pl.multiple_of(i * 16, 16)
            pos = base + c * CHUNK + off + lax.iota(jnp.int32, 16)
            fld = lax.shift_right_logical(pos, LOG2_B)
            v = idx_bufs[k][pl.ds(off, 16)] + fld * V
            idx_bufs[k][pl.ds(off, 16)] = lax.shift_right_logical(v, 2)
            return 0

        lax.fori_loop(0, CHUNK // 16, fix, 0)

    def drain(c):
        k = c % 2
        pltpu.sync_copy(wide_bufs[k],
                        out_hbm.at[pl.ds(base + c * CHUNK, CHUNK)])

    prev = None
    for c in range(NCHUNK):
        k = c % 2
        stage_indices(c, k)
        cp = pltpu.async_copy(tables_hbm.at[idx_bufs[k]], wide_bufs[k], sems[k])
        if prev is not None:
            prev.wait()
            drain(c - 1)
        prev = cp
    prev.wait()
    drain(NCHUNK - 1)


@functools.lru_cache(maxsize=None)
def _sc_gather():
    return pl.kernel(
        _sc_gather_body,
        mesh=plsc.VectorSubcoreMesh(core_axis_name="c", subcore_axis_name="s"),
        out_type=jax.ShapeDtypeStruct((TOTAL_ROWS, 128), jnp.float32),
        scratch_types=[
            pltpu.VMEM((CHUNK,), jnp.int32),
            pltpu.VMEM((CHUNK,), jnp.int32),
            pltpu.VMEM((CHUNK, 128), jnp.float32),
            pltpu.VMEM((CHUNK, 128), jnp.float32),
            pltpu.SemaphoreType.DMA,
            pltpu.SemaphoreType.DMA,
        ],
        compiler_params=pltpu.CompilerParams(use_tc_tiling_on_sc=False),
    )


TV = 4096


def _transpose_body(xt_ref, out_ref):
    xt = xt_ref[0].T.reshape(TV // 4, 4, D)
    out_ref[0] = jnp.concatenate([xt[:, j, :] for j in range(4)], axis=1)


def _transpose_tables(tables_t):
    return pl.pallas_call(
        _transpose_body,
        grid=(F, V // TV + 1),
        in_specs=[pl.BlockSpec((1, D, TV), lambda f, i: (f, 0, i))],
        out_specs=pl.BlockSpec((1, TV // 4, 128), lambda f, i: (f, i, 0)),
        out_shape=jax.ShapeDtypeStruct((F, V // 4, 128), jnp.float32),
    )(tables_t)


BT = 1024


def _mlp_body(idx_ref, embs_ref, w1_ref, b1_ref, w2_ref, b2_ref, out_ref):
    lane_group = lax.broadcasted_iota(jnp.int32, (BT, 128), 1) // D
    acc = jnp.zeros((BT, D), jnp.float32)
    for f in range(F):
        q = (idx_ref[f] & 3).reshape(BT, 1)
        xm = jnp.where(lane_group == q, embs_ref[f], 0.0)
        acc = acc + jnp.dot(xm, w1_ref[f],
                            preferred_element_type=jnp.float32)
    h = jnp.maximum(acc + b1_ref[...], 0.0)
    out_ref[...] = jnp.dot(h, w2_ref[...],
                           preferred_element_type=jnp.float32) + b2_ref[...]


def _mlp(indices, embs_wide, W1s, b1r, W2, b2r):
    return pl.pallas_call(
        _mlp_body,
        grid=(B // BT,),
        in_specs=[
            pl.BlockSpec((F, BT), lambda i: (0, i)),
            pl.BlockSpec((F, BT, 128), lambda i: (0, i, 0)),
            pl.BlockSpec((F, 128, D), lambda i: (0, 0, 0)),
            pl.BlockSpec((1, D), lambda i: (0, 0)),
            pl.BlockSpec((D, D), lambda i: (0, 0)),
            pl.BlockSpec((1, D), lambda i: (0, 0)),
        ],
        out_specs=pl.BlockSpec((BT, D), lambda i: (i, 0)),
        out_shape=jax.ShapeDtypeStruct((B, D), jnp.float32),
    )(indices, embs_wide, W1s, b1r, W2, b2r)


def kernel(indices, tables, W1, b1, W2, b2):
    tables_t = tables.transpose(0, 2, 1)
    tables_wide = _transpose_tables(tables_t).reshape(F * V * D // 128, 128)
    idx_flat = indices.reshape(TOTAL_ROWS)
    embs = _sc_gather()(tables_wide, idx_flat)
    embs_wide = embs.reshape(F, B, 128)

    W1s = jnp.tile(W1.reshape(F, D, D), (1, 128 // D, 1))
    return _mlp(indices, embs_wide, W1s, b1.reshape(1, D), W2,
                b2.reshape(1, D))
